# baseline (speedup 1.0000x reference)
import functools

import jax
import jax.numpy as jnp
from jax import lax
from jax.experimental import pallas as pl
from jax.experimental.pallas import tpu as pltpu
from jax.experimental.pallas import tpu_sc as plsc

N = 10000
E = 320000
NUM_OP = 32
NUM_ET = 8
VOCAB = 100000
TEXT = 64
HID = 128
OUT = 512
B = 64
L = 128

BR = 1000
NROW = 10112
C_E = 128
G_E = 16
NGRP = 5
EPT = C_E * G_E * NGRP
EPAD = 32 * EPT


def _prep_body(xp_ref, ope_ref, ee_ref, wl0_ref, bl0_ref, wl1_ref, bl1_ref,
               wl2_ref, bl2_ref, h0_ref, r0_ref, et0_ref, et1_ref, et2_ref):
    op_ids = xp_ref[:, 0:1].astype(jnp.int32)
    iota = lax.broadcasted_iota(jnp.int32, (BR, NUM_OP), 1)
    oh = (iota == op_ids).astype(jnp.float32)
    emb = jnp.dot(oh, ope_ref[...], preferred_element_type=jnp.float32)
    h0 = jnp.concatenate(
        [emb, xp_ref[:, 1:13], jnp.zeros((BR, HID - 44), jnp.float32)], axis=1)
    h0_ref[...] = h0
    et0 = jnp.dot(ee_ref[...], wl0_ref[...],
                  preferred_element_type=jnp.float32) + bl0_ref[...]
    et1 = jnp.dot(ee_ref[...], wl1_ref[...],
                  preferred_element_type=jnp.float32) + bl1_ref[...]
    et2 = jnp.dot(ee_ref[...], wl2_ref[...],
                  preferred_element_type=jnp.float32) + bl2_ref[...]
    et0_ref[...] = et0
    et1_ref[...] = et1
    et2_ref[...] = et2
    for t in range(NUM_ET):
        r0_ref[:, t, :] = jnp.maximum(h0 + et0[t:t + 1, :], 0.0)


def _prep(xp, ope, ee, wl0, bl0, wl1, bl1, wl2, bl2):
    grid = (N // BR,)
    full = lambda shp: pl.BlockSpec(shp, lambda i: (0,) * len(shp))
    return pl.pallas_call(
        _prep_body,
        grid=grid,
        in_specs=[
            pl.BlockSpec((BR, 16), lambda i: (i, 0)),
            full((NUM_OP, 32)), full((NUM_ET, 16)),
            full((16, HID)), full((1, HID)),
            full((16, HID)), full((1, HID)),
            full((16, HID)), full((1, HID)),
        ],
        out_specs=[
            pl.BlockSpec((BR, HID), lambda i: (i, 0)),
            pl.BlockSpec((BR, NUM_ET, HID), lambda i: (i, 0, 0)),
            full((NUM_ET, HID)), full((NUM_ET, HID)), full((NUM_ET, HID)),
        ],
        out_shape=[
            jax.ShapeDtypeStruct((N, HID), jnp.float32),
            jax.ShapeDtypeStruct((N, NUM_ET, HID), jnp.float32),
            jax.ShapeDtypeStruct((NUM_ET, HID), jnp.float32),
            jax.ShapeDtypeStruct((NUM_ET, HID), jnp.float32),
            jax.ShapeDtypeStruct((NUM_ET, HID), jnp.float32),
        ],
    )(xp, ope, ee, wl0, bl0, wl1, bl1, wl2, bl2)


def _ridx_body(src_ref, ea_ref, out_ref):
    out_ref[...] = src_ref[...] * NUM_ET + ea_ref[...]


def _ridx(src_pad, ea_pad):
    s2 = src_pad.reshape(EPAD // 128, 128)
    e2 = ea_pad.reshape(EPAD // 128, 128)
    out = pl.pallas_call(
        _ridx_body,
        out_shape=jax.ShapeDtypeStruct((EPAD // 128, 128), jnp.int32),
    )(s2, e2)
    return out


def _layer_body(din, residual, emit_r, *refs):
    if emit_r:
        (p_ref, h_ref, w1_ref, b1_ref, w2_ref, b2_ref, eps_ref, et_ref,
         hn_ref, r_ref) = refs
    else:
        (p_ref, h_ref, w1_ref, b1_ref, w2_ref, b2_ref, eps_ref,
         hn_ref) = refs
    h = h_ref[...]
    agg = p_ref[0] + p_ref[1]
    out = agg + (1.0 + eps_ref[0, 0]) * h
    hid = jnp.maximum(
        jnp.dot(out, w1_ref[...], preferred_element_type=jnp.float32)
        + b1_ref[...], 0.0)
    out2 = jnp.dot(hid, w2_ref[...],
                   preferred_element_type=jnp.float32) + b2_ref[...]
    mu = jnp.mean(out2, axis=-1, keepdims=True)
    var = jnp.mean((out2 - mu) ** 2, axis=-1, keepdims=True)
    out2 = (out2 - mu) * lax.rsqrt(var + 1e-5)
    if residual:
        out2 = out2 + h
    hn = jnp.where(out2 >= 0.0, out2, 0.1 * out2)
    hn_ref[...] = hn
    if emit_r:
        et = et_ref[...]
        for t in range(NUM_ET):
            r_ref[:, t, :] = jnp.maximum(hn + et[t:t + 1, :], 0.0)


def _layer(p, h, w1, b1, w2, b2, eps, et_next, din, residual):
    emit_r = et_next is not None
    grid = (N // BR,)
    full = lambda shp: pl.BlockSpec(shp, lambda i: (0,) * len(shp))
    in_specs = [
        pl.BlockSpec((2, BR, din), lambda i: (0, i, 0)),
        pl.BlockSpec((BR, din), lambda i: (i, 0)),
        full((din, HID)), full((1, HID)), full((HID, HID)), full((1, HID)),
        full((1, 1)),
    ]
    out_specs = [pl.BlockSpec((BR, HID), lambda i: (i, 0))]
    out_shape = [jax.ShapeDtypeStruct((N, HID), jnp.float32)]
    args = [p, h, w1, b1, w2, b2, eps]
    if emit_r:
        in_specs.append(full((NUM_ET, HID)))
        out_specs.append(pl.BlockSpec((BR, NUM_ET, HID), lambda i: (i, 0, 0)))
        out_shape.append(
            jax.ShapeDtypeStruct((N, NUM_ET, HID), jnp.float32))
        args.append(et_next)
    res = pl.pallas_call(
        functools.partial(_layer_body, din, residual, emit_r),
        grid=grid,
        in_specs=in_specs,
        out_specs=out_specs,
        out_shape=out_shape,
    )(*args)
    return res if emit_r else (res[0], None)


def _final_body(h3_ref, b8_ref, x45_ref, tp_ref, w1a_ref, w1b_ref, w1c_ref,
                b1_ref, w2_ref, b2_ref, out_ref):
    brow = b8_ref[0:1, :]
    iota = lax.broadcasted_iota(jnp.int32, (B, N), 0)
    oh = (iota == brow).astype(jnp.float32)
    g = jnp.dot(oh, h3_ref[...], preferred_element_type=jnp.float32)
    sums = jnp.dot(oh, x45_ref[...], preferred_element_type=jnp.float32)
    counts = jnp.sum(oh, axis=1, keepdims=True)
    denom = jnp.maximum(counts, 1.0)
    gs = jnp.concatenate(
        [counts, sums[:, 1:2] / denom, sums[:, 0:1] / denom,
         jnp.zeros((B, 5), jnp.float32)], axis=1)
    text = ((tp_ref[0] + tp_ref[1]) * (1.0 / L))[:, :TEXT]
    hid = (jnp.dot(g, w1a_ref[...], preferred_element_type=jnp.float32)
           + jnp.dot(gs, w1b_ref[...], preferred_element_type=jnp.float32)
           + jnp.dot(text, w1c_ref[...], preferred_element_type=jnp.float32)
           + b1_ref[...])
    hid = jnp.where(hid >= 0.0, hid, 0.1 * hid)
    out_ref[...] = jnp.dot(
        hid, w2_ref[...], preferred_element_type=jnp.float32) + b2_ref[...]


def _final(h3, b8, x45, tpart, w1a, w1b8, w1c, b1, w2, b2):
    return pl.pallas_call(
        _final_body,
        out_shape=jax.ShapeDtypeStruct((B, OUT), jnp.float32),
    )(h3, b8, x45, tpart, w1a, w1b8, w1c, b1, w2, b2)


NSUB = 16
K_CH = EPT // C_E
ROWS_PT = NROW // NSUB


def _edge_pass(r_tab, ridx3, dst3, zeros_nd, din):
    rf = r_tab.reshape(N * NUM_ET, din)
    mesh = plsc.VectorSubcoreMesh(core_axis_name="c", subcore_axis_name="s")

    @functools.partial(
        pl.kernel,
        out_type=jax.ShapeDtypeStruct((2, NROW, din), jnp.float32),
        mesh=mesh,
        scratch_types=[
            pltpu.VMEM((K_CH, C_E), jnp.int32),
            pltpu.VMEM((K_CH, C_E), jnp.int32),
            pltpu.VMEM((C_E, din), jnp.float32),
            pltpu.VMEM_SHARED((NROW, din), jnp.float32),
            pltpu.SemaphoreType.DMA,
        ],
    )
    def k(r_hbm, ridx_hbm, dst_hbm, z_hbm, out_hbm,
          ridx_v, dst_v, msg_v, acc, sem):
        c = lax.axis_index("c")
        s = lax.axis_index("s")
        wid = c * NSUB + s
        pltpu.sync_copy(z_hbm.at[pl.ds(s * ROWS_PT, ROWS_PT)],
                        acc.at[pl.ds(s * ROWS_PT, ROWS_PT)])
        pltpu.sync_copy(ridx_hbm.at[wid], ridx_v)
        pltpu.sync_copy(dst_hbm.at[wid], dst_v)
        plsc.subcore_barrier()

        def body(j, carry):
            pltpu.async_copy(r_hbm.at[ridx_v.at[j]], msg_v, sem).wait()
            pltpu.sync_copy(msg_v, acc.at[dst_v.at[j]], add=True)
            return carry

        lax.fori_loop(0, K_CH, body, 0)
        plsc.subcore_barrier()
        pltpu.sync_copy(acc.at[pl.ds(s * ROWS_PT, ROWS_PT)],
                        out_hbm.at[c].at[pl.ds(s * ROWS_PT, ROWS_PT)])

    return k(rf, ridx3, dst3, zeros_nd)


TOK_CH = (B * L) // (32 * 128)
BROWS_PT = 8


def _text_pass(token_embed, ids3, rep3, zeros_b):
    mesh = plsc.VectorSubcoreMesh(core_axis_name="c", subcore_axis_name="s")

    @functools.partial(
        pl.kernel,
        out_type=jax.ShapeDtypeStruct((2, B, HID), jnp.float32),
        mesh=mesh,
        scratch_types=[
            pltpu.VMEM((TOK_CH, 128), jnp.int32),
            pltpu.VMEM((TOK_CH, 128), jnp.int32),
            pltpu.VMEM((128, HID), jnp.float32),
            pltpu.VMEM_SHARED((B, HID), jnp.float32),
            pltpu.SemaphoreType.DMA,
        ],
    )
    def k(tok_hbm, ids_hbm, rep_hbm, z_hbm, out_hbm,
          ids_v, rep_v, msg_v, acc, sem):
        c = lax.axis_index("c")
        s = lax.axis_index("s")
        wid = c * NSUB + s

        @pl.when(s < B // BROWS_PT)
        def _():
            pltpu.sync_copy(z_hbm.at[pl.ds(s * BROWS_PT, BROWS_PT)],
                            acc.at[pl.ds(s * BROWS_PT, BROWS_PT)])

        pltpu.sync_copy(ids_hbm.at[wid], ids_v)
        pltpu.sync_copy(rep_hbm.at[wid], rep_v)
        plsc.subcore_barrier()
        for j in range(TOK_CH):
            pltpu.async_copy(tok_hbm.at[ids_v.at[j]], msg_v, sem).wait()
            pltpu.sync_copy(msg_v, acc.at[rep_v.at[j]], add=True)
        plsc.subcore_barrier()

        @pl.when(s < B // BROWS_PT)
        def _():
            pltpu.sync_copy(acc.at[pl.ds(s * BROWS_PT, BROWS_PT)],
                            out_hbm.at[c].at[pl.ds(s * BROWS_PT, BROWS_PT)])

    return k(token_embed, ids3, rep3, zeros_b)


def kernel(x, sql_mask, params, edge_index, edge_attr, batch, sql_ids):
    convs = params['convs']
    xp = jnp.pad(x, ((0, 0), (0, 3)))
    src = edge_index[0].astype(jnp.int32)
    dst = edge_index[1].astype(jnp.int32)
    ea = edge_attr.astype(jnp.int32)
    src_pad = jnp.pad(src, (0, EPAD - E))
    ea_pad = jnp.pad(ea, (0, EPAD - E))
    dst_pad = jnp.pad(dst, (0, EPAD - E), constant_values=N)

    wl0 = jnp.pad(convs[0]['Wl'], ((0, 0), (0, HID - 44)))
    bl0 = jnp.pad(convs[0]['bl'], (0, HID - 44)).reshape(1, HID)
    w1_0 = jnp.pad(convs[0]['W1'], ((0, HID - 44), (0, 0)))

    h0, r0, et0, et1, et2 = _prep(
        xp, params['op_embed'], params['edge_embed'],
        wl0, bl0,
        convs[1]['Wl'], convs[1]['bl'].reshape(1, HID),
        convs[2]['Wl'], convs[2]['bl'].reshape(1, HID))

    ridx3 = _ridx(src_pad, ea_pad).reshape(32, K_CH, C_E)
    dst3 = dst_pad.reshape(32, K_CH, C_E)

    ets = [et0, et1, et2]
    h = h0
    r_tab = r0
    dins = [HID, HID, HID]
    w1s = [w1_0, convs[1]['W1'], convs[2]['W1']]
    zeros_nd = jnp.zeros((NROW, HID), jnp.float32)
    for l in range(3):
        c = convs[l]
        p = _edge_pass(r_tab, ridx3, dst3, zeros_nd, dins[l])
        et_next = ets[l + 1] if l < 2 else None
        h, r_tab = _layer(
            p, h, w1s[l], c['b1'].reshape(1, HID),
            c['W2'], c['b2'].reshape(1, HID),
            c['eps'].reshape(1, 1), et_next, dins[l], residual=(l > 0))

    ids3 = sql_ids.astype(jnp.int32).reshape(32, TOK_CH, 128)
    rep3 = jnp.broadcast_to(
        jnp.arange(B, dtype=jnp.int32)[:, None], (B, L)).reshape(
            32, TOK_CH, 128)
    tok_pad = jnp.pad(params['token_embed'], ((0, 0), (0, HID - TEXT)))
    tpart = _text_pass(tok_pad, ids3, rep3,
                       jnp.zeros((B, HID), jnp.float32))

    b8 = jnp.broadcast_to(batch.astype(jnp.int32).reshape(1, N), (8, N))
    x45 = x[:, 4:6]
    mlp = params['mlp']
    w1a = mlp['W1'][:HID]
    w1b8 = jnp.pad(mlp['W1'][HID:HID + 3], ((0, 5), (0, 0)))
    w1c = mlp['W1'][HID + 3:]
    return _final(h, b8, x45, tpart, w1a, w1b8, w1c,
                  mlp['b1'].reshape(1, HID), mlp['W2'],
                  mlp['b2'].reshape(1, OUT))


# speedup vs baseline: 1.1822x; 1.1822x over previous
import functools

import jax
import jax.numpy as jnp
from jax import lax
from jax.experimental import pallas as pl
from jax.experimental.pallas import tpu as pltpu
from jax.experimental.pallas import tpu_sc as plsc

N = 10000
E = 320000
NUM_OP = 32
NUM_ET = 8
VOCAB = 100000
TEXT = 64
HID = 128
OUT = 512
B = 64
L = 128

BR = 1000
NROW = 10112
C_E = 128
EPT = 10112
EPAD = 32 * EPT


def _prep_body(xp_ref, ope_ref, ee_ref, wl0_ref, bl0_ref, wl1_ref, bl1_ref,
               wl2_ref, bl2_ref, h0_ref, r0_ref, et0_ref, et1_ref, et2_ref):
    op_ids = xp_ref[:, 0:1].astype(jnp.int32)
    iota = lax.broadcasted_iota(jnp.int32, (BR, NUM_OP), 1)
    oh = (iota == op_ids).astype(jnp.float32)
    emb = jnp.dot(oh, ope_ref[...], preferred_element_type=jnp.float32)
    h0 = jnp.concatenate(
        [emb, xp_ref[:, 1:13], jnp.zeros((BR, HID - 44), jnp.float32)], axis=1)
    h0_ref[...] = h0
    et0 = jnp.dot(ee_ref[...], wl0_ref[...],
                  preferred_element_type=jnp.float32) + bl0_ref[...]
    et1 = jnp.dot(ee_ref[...], wl1_ref[...],
                  preferred_element_type=jnp.float32) + bl1_ref[...]
    et2 = jnp.dot(ee_ref[...], wl2_ref[...],
                  preferred_element_type=jnp.float32) + bl2_ref[...]
    et0_ref[...] = et0
    et1_ref[...] = et1
    et2_ref[...] = et2
    for t in range(NUM_ET):
        r0_ref[:, t, :] = jnp.maximum(h0 + et0[t:t + 1, :], 0.0)


def _prep(xp, ope, ee, wl0, bl0, wl1, bl1, wl2, bl2):
    grid = (N // BR,)
    full = lambda shp: pl.BlockSpec(shp, lambda i: (0,) * len(shp))
    return pl.pallas_call(
        _prep_body,
        grid=grid,
        in_specs=[
            pl.BlockSpec((BR, 16), lambda i: (i, 0)),
            full((NUM_OP, 32)), full((NUM_ET, 16)),
            full((16, HID)), full((1, HID)),
            full((16, HID)), full((1, HID)),
            full((16, HID)), full((1, HID)),
        ],
        out_specs=[
            pl.BlockSpec((BR, HID), lambda i: (i, 0)),
            pl.BlockSpec((BR, NUM_ET, HID), lambda i: (i, 0, 0)),
            full((NUM_ET, HID)), full((NUM_ET, HID)), full((NUM_ET, HID)),
        ],
        out_shape=[
            jax.ShapeDtypeStruct((N, HID), jnp.float32),
            jax.ShapeDtypeStruct((N, NUM_ET, HID), jnp.float32),
            jax.ShapeDtypeStruct((NUM_ET, HID), jnp.float32),
            jax.ShapeDtypeStruct((NUM_ET, HID), jnp.float32),
            jax.ShapeDtypeStruct((NUM_ET, HID), jnp.float32),
        ],
    )(xp, ope, ee, wl0, bl0, wl1, bl1, wl2, bl2)


def _ridx_body(src_ref, ea_ref, out_ref):
    out_ref[...] = src_ref[...] * NUM_ET + ea_ref[...]


def _ridx(src_pad, ea_pad):
    s2 = src_pad.reshape(EPAD // 128, 128)
    e2 = ea_pad.reshape(EPAD // 128, 128)
    out = pl.pallas_call(
        _ridx_body,
        out_shape=jax.ShapeDtypeStruct((EPAD // 128, 128), jnp.int32),
    )(s2, e2)
    return out


def _layer_body(din, residual, emit_r, *refs):
    if emit_r:
        (p_ref, h_ref, w1_ref, b1_ref, w2_ref, b2_ref, eps_ref, et_ref,
         hn_ref, r_ref) = refs
    else:
        (p_ref, h_ref, w1_ref, b1_ref, w2_ref, b2_ref, eps_ref,
         hn_ref) = refs
    h = h_ref[...]
    agg = p_ref[0] + p_ref[1]
    out = agg + (1.0 + eps_ref[0, 0]) * h
    hid = jnp.maximum(
        jnp.dot(out, w1_ref[...], preferred_element_type=jnp.float32)
        + b1_ref[...], 0.0)
    out2 = jnp.dot(hid, w2_ref[...],
                   preferred_element_type=jnp.float32) + b2_ref[...]
    mu = jnp.mean(out2, axis=-1, keepdims=True)
    var = jnp.mean((out2 - mu) ** 2, axis=-1, keepdims=True)
    out2 = (out2 - mu) * lax.rsqrt(var + 1e-5)
    if residual:
        out2 = out2 + h
    hn = jnp.where(out2 >= 0.0, out2, 0.1 * out2)
    hn_ref[...] = hn
    if emit_r:
        et = et_ref[...]
        for t in range(NUM_ET):
            r_ref[:, t, :] = jnp.maximum(hn + et[t:t + 1, :], 0.0)


def _layer(p, h, w1, b1, w2, b2, eps, et_next, din, residual):
    emit_r = et_next is not None
    grid = (N // BR,)
    full = lambda shp: pl.BlockSpec(shp, lambda i: (0,) * len(shp))
    in_specs = [
        pl.BlockSpec((2, BR, din), lambda i: (0, i, 0)),
        pl.BlockSpec((BR, din), lambda i: (i, 0)),
        full((din, HID)), full((1, HID)), full((HID, HID)), full((1, HID)),
        full((1, 1)),
    ]
    out_specs = [pl.BlockSpec((BR, HID), lambda i: (i, 0))]
    out_shape = [jax.ShapeDtypeStruct((N, HID), jnp.float32)]
    args = [p, h, w1, b1, w2, b2, eps]
    if emit_r:
        in_specs.append(full((NUM_ET, HID)))
        out_specs.append(pl.BlockSpec((BR, NUM_ET, HID), lambda i: (i, 0, 0)))
        out_shape.append(
            jax.ShapeDtypeStruct((N, NUM_ET, HID), jnp.float32))
        args.append(et_next)
    res = pl.pallas_call(
        functools.partial(_layer_body, din, residual, emit_r),
        grid=grid,
        in_specs=in_specs,
        out_specs=out_specs,
        out_shape=out_shape,
    )(*args)
    return res if emit_r else (res[0], None)


def _final_body(h3_ref, b8_ref, x45_ref, tp_ref, w1a_ref, w1b_ref, w1c_ref,
                b1_ref, w2_ref, b2_ref, out_ref):
    brow = b8_ref[0:1, :]
    iota = lax.broadcasted_iota(jnp.int32, (B, N), 0)
    oh = (iota == brow).astype(jnp.float32)
    g = jnp.dot(oh, h3_ref[...], preferred_element_type=jnp.float32)
    sums = jnp.dot(oh, x45_ref[...], preferred_element_type=jnp.float32)
    counts = jnp.sum(oh, axis=1, keepdims=True)
    denom = jnp.maximum(counts, 1.0)
    gs = jnp.concatenate(
        [counts, sums[:, 1:2] / denom, sums[:, 0:1] / denom,
         jnp.zeros((B, 5), jnp.float32)], axis=1)
    text = ((tp_ref[0] + tp_ref[1]) * (1.0 / L))[:, :TEXT]
    hid = (jnp.dot(g, w1a_ref[...], preferred_element_type=jnp.float32)
           + jnp.dot(gs, w1b_ref[...], preferred_element_type=jnp.float32)
           + jnp.dot(text, w1c_ref[...], preferred_element_type=jnp.float32)
           + b1_ref[...])
    hid = jnp.where(hid >= 0.0, hid, 0.1 * hid)
    out_ref[...] = jnp.dot(
        hid, w2_ref[...], preferred_element_type=jnp.float32) + b2_ref[...]


def _final(h3, b8, x45, tpart, w1a, w1b8, w1c, b1, w2, b2):
    return pl.pallas_call(
        _final_body,
        out_shape=jax.ShapeDtypeStruct((B, OUT), jnp.float32),
    )(h3, b8, x45, tpart, w1a, w1b8, w1c, b1, w2, b2)


NSUB = 16
K_CH = EPT // C_E
ROWS_PT = NROW // NSUB


def _edge_pass(r_tab, ridx3, dst3, zeros_nd, din):
    rf = r_tab.reshape(N * NUM_ET, din)
    mesh = plsc.VectorSubcoreMesh(core_axis_name="c", subcore_axis_name="s")

    @functools.partial(
        pl.kernel,
        out_type=jax.ShapeDtypeStruct((2, NROW, din), jnp.float32),
        mesh=mesh,
        scratch_types=[
            pltpu.VMEM((K_CH, C_E), jnp.int32),
            pltpu.VMEM((K_CH, C_E), jnp.int32),
            pltpu.VMEM((C_E, din), jnp.float32),
            pltpu.VMEM_SHARED((NROW, din), jnp.float32),
            pltpu.SemaphoreType.DMA,
        ],
    )
    def k(r_hbm, ridx_hbm, dst_hbm, z_hbm, out_hbm,
          ridx_v, dst_v, msg_v, acc, sem):
        c = lax.axis_index("c")
        s = lax.axis_index("s")
        wid = c * NSUB + s
        pltpu.sync_copy(z_hbm.at[pl.ds(s * ROWS_PT, ROWS_PT)],
                        acc.at[pl.ds(s * ROWS_PT, ROWS_PT)])
        pltpu.sync_copy(ridx_hbm.at[wid], ridx_v)
        pltpu.sync_copy(dst_hbm.at[wid], dst_v)
        plsc.subcore_barrier()

        def body(j, carry):
            pltpu.async_copy(r_hbm.at[ridx_v.at[j]], msg_v, sem).wait()
            pltpu.sync_copy(msg_v, acc.at[dst_v.at[j]], add=True)
            return carry

        lax.fori_loop(0, K_CH, body, 0)
        plsc.subcore_barrier()
        pltpu.sync_copy(acc.at[pl.ds(s * ROWS_PT, ROWS_PT)],
                        out_hbm.at[c].at[pl.ds(s * ROWS_PT, ROWS_PT)])

    return k(rf, ridx3, dst3, zeros_nd)


TOK_CH = (B * L) // (32 * 128)
BROWS_PT = 8


def _text_pass(token_embed, ids3, rep3, zeros_b):
    mesh = plsc.VectorSubcoreMesh(core_axis_name="c", subcore_axis_name="s")

    @functools.partial(
        pl.kernel,
        out_type=jax.ShapeDtypeStruct((2, B, HID), jnp.float32),
        mesh=mesh,
        scratch_types=[
            pltpu.VMEM((TOK_CH, 128), jnp.int32),
            pltpu.VMEM((TOK_CH, 128), jnp.int32),
            pltpu.VMEM((128, HID), jnp.float32),
            pltpu.VMEM_SHARED((B, HID), jnp.float32),
            pltpu.SemaphoreType.DMA,
        ],
    )
    def k(tok_hbm, ids_hbm, rep_hbm, z_hbm, out_hbm,
          ids_v, rep_v, msg_v, acc, sem):
        c = lax.axis_index("c")
        s = lax.axis_index("s")
        wid = c * NSUB + s

        @pl.when(s < B // BROWS_PT)
        def _():
            pltpu.sync_copy(z_hbm.at[pl.ds(s * BROWS_PT, BROWS_PT)],
                            acc.at[pl.ds(s * BROWS_PT, BROWS_PT)])

        pltpu.sync_copy(ids_hbm.at[wid], ids_v)
        pltpu.sync_copy(rep_hbm.at[wid], rep_v)
        plsc.subcore_barrier()
        for j in range(TOK_CH):
            pltpu.async_copy(tok_hbm.at[ids_v.at[j]], msg_v, sem).wait()
            pltpu.sync_copy(msg_v, acc.at[rep_v.at[j]], add=True)
        plsc.subcore_barrier()

        @pl.when(s < B // BROWS_PT)
        def _():
            pltpu.sync_copy(acc.at[pl.ds(s * BROWS_PT, BROWS_PT)],
                            out_hbm.at[c].at[pl.ds(s * BROWS_PT, BROWS_PT)])

    return k(token_embed, ids3, rep3, zeros_b)


def kernel(x, sql_mask, params, edge_index, edge_attr, batch, sql_ids):
    convs = params['convs']
    xp = jnp.pad(x, ((0, 0), (0, 3)))
    src = edge_index[0].astype(jnp.int32)
    dst = edge_index[1].astype(jnp.int32)
    ea = edge_attr.astype(jnp.int32)
    src_pad = jnp.pad(src, (0, EPAD - E))
    ea_pad = jnp.pad(ea, (0, EPAD - E))
    dst_pad = jnp.pad(dst, (0, EPAD - E), constant_values=N)

    wl0 = jnp.pad(convs[0]['Wl'], ((0, 0), (0, HID - 44)))
    bl0 = jnp.pad(convs[0]['bl'], (0, HID - 44)).reshape(1, HID)
    w1_0 = jnp.pad(convs[0]['W1'], ((0, HID - 44), (0, 0)))

    h0, r0, et0, et1, et2 = _prep(
        xp, params['op_embed'], params['edge_embed'],
        wl0, bl0,
        convs[1]['Wl'], convs[1]['bl'].reshape(1, HID),
        convs[2]['Wl'], convs[2]['bl'].reshape(1, HID))

    ridx3 = _ridx(src_pad, ea_pad).reshape(32, K_CH, C_E)
    dst3 = dst_pad.reshape(32, K_CH, C_E)

    ets = [et0, et1, et2]
    h = h0
    r_tab = r0
    dins = [HID, HID, HID]
    w1s = [w1_0, convs[1]['W1'], convs[2]['W1']]
    zeros_nd = jnp.zeros((NROW, HID), jnp.float32)
    for l in range(3):
        c = convs[l]
        p = _edge_pass(r_tab, ridx3, dst3, zeros_nd, dins[l])
        et_next = ets[l + 1] if l < 2 else None
        h, r_tab = _layer(
            p, h, w1s[l], c['b1'].reshape(1, HID),
            c['W2'], c['b2'].reshape(1, HID),
            c['eps'].reshape(1, 1), et_next, dins[l], residual=(l > 0))

    ids3 = sql_ids.astype(jnp.int32).reshape(32, TOK_CH, 128)
    rep3 = jnp.broadcast_to(
        jnp.arange(B, dtype=jnp.int32)[:, None], (B, L)).reshape(
            32, TOK_CH, 128)
    tok_pad = jnp.pad(params['token_embed'], ((0, 0), (0, HID - TEXT)))
    tpart = _text_pass(tok_pad, ids3, rep3,
                       jnp.zeros((B, HID), jnp.float32))

    b8 = jnp.broadcast_to(batch.astype(jnp.int32).reshape(1, N), (8, N))
    x45 = x[:, 4:6]
    mlp = params['mlp']
    w1a = mlp['W1'][:HID]
    w1b8 = jnp.pad(mlp['W1'][HID:HID + 3], ((0, 5), (0, 0)))
    w1c = mlp['W1'][HID + 3:]
    return _final(h, b8, x45, tpart, w1a, w1b8, w1c,
                  mlp['b1'].reshape(1, HID), mlp['W2'],
                  mlp['b2'].reshape(1, OUT))


# speedup vs baseline: 2.0836x; 1.7624x over previous
import functools

import jax
import jax.numpy as jnp
from jax import lax
from jax.experimental import pallas as pl
from jax.experimental.pallas import tpu as pltpu
from jax.experimental.pallas import tpu_sc as plsc

N = 10000
E = 320000
NUM_OP = 32
NUM_ET = 8
VOCAB = 100000
TEXT = 64
HID = 128
OUT = 512
B = 64
L = 128

BR = 1000
NROW = 10112
C_E = 128
EPT = 10112
EPAD = 32 * EPT


def _prep_body(xp_ref, ope_ref, ee_ref, wl0_ref, bl0_ref, wl1_ref, bl1_ref,
               wl2_ref, bl2_ref, h0_ref, r0_ref, et0_ref, et1_ref, et2_ref):
    op_ids = xp_ref[:, 0:1].astype(jnp.int32)
    iota = lax.broadcasted_iota(jnp.int32, (BR, NUM_OP), 1)
    oh = (iota == op_ids).astype(jnp.float32)
    emb = jnp.dot(oh, ope_ref[...], preferred_element_type=jnp.float32)
    h0 = jnp.concatenate(
        [emb, xp_ref[:, 1:13], jnp.zeros((BR, HID - 44), jnp.float32)], axis=1)
    h0_ref[...] = h0
    et0 = jnp.dot(ee_ref[...], wl0_ref[...],
                  preferred_element_type=jnp.float32) + bl0_ref[...]
    et1 = jnp.dot(ee_ref[...], wl1_ref[...],
                  preferred_element_type=jnp.float32) + bl1_ref[...]
    et2 = jnp.dot(ee_ref[...], wl2_ref[...],
                  preferred_element_type=jnp.float32) + bl2_ref[...]
    et0_ref[...] = et0
    et1_ref[...] = et1
    et2_ref[...] = et2
    for t in range(NUM_ET):
        r0_ref[:, t, :] = jnp.maximum(h0 + et0[t:t + 1, :], 0.0)


def _prep(xp, ope, ee, wl0, bl0, wl1, bl1, wl2, bl2):
    grid = (N // BR,)
    full = lambda shp: pl.BlockSpec(shp, lambda i: (0,) * len(shp))
    return pl.pallas_call(
        _prep_body,
        grid=grid,
        in_specs=[
            pl.BlockSpec((BR, 16), lambda i: (i, 0)),
            full((NUM_OP, 32)), full((NUM_ET, 16)),
            full((16, HID)), full((1, HID)),
            full((16, HID)), full((1, HID)),
            full((16, HID)), full((1, HID)),
        ],
        out_specs=[
            pl.BlockSpec((BR, HID), lambda i: (i, 0)),
            pl.BlockSpec((BR, NUM_ET, HID), lambda i: (i, 0, 0)),
            full((NUM_ET, HID)), full((NUM_ET, HID)), full((NUM_ET, HID)),
        ],
        out_shape=[
            jax.ShapeDtypeStruct((N, HID), jnp.float32),
            jax.ShapeDtypeStruct((N, NUM_ET, HID), jnp.float32),
            jax.ShapeDtypeStruct((NUM_ET, HID), jnp.float32),
            jax.ShapeDtypeStruct((NUM_ET, HID), jnp.float32),
            jax.ShapeDtypeStruct((NUM_ET, HID), jnp.float32),
        ],
    )(xp, ope, ee, wl0, bl0, wl1, bl1, wl2, bl2)


def _ridx_body(src_ref, ea_ref, out_ref):
    out_ref[...] = src_ref[...] * NUM_ET + ea_ref[...]


def _ridx(src_pad, ea_pad):
    s2 = src_pad.reshape(EPAD // 128, 128)
    e2 = ea_pad.reshape(EPAD // 128, 128)
    out = pl.pallas_call(
        _ridx_body,
        out_shape=jax.ShapeDtypeStruct((EPAD // 128, 128), jnp.int32),
    )(s2, e2)
    return out


def _layer_body(din, residual, emit_r, *refs):
    if emit_r:
        (p_ref, h_ref, w1_ref, b1_ref, w2_ref, b2_ref, eps_ref, et_ref,
         hn_ref, r_ref) = refs
    else:
        (p_ref, h_ref, w1_ref, b1_ref, w2_ref, b2_ref, eps_ref,
         hn_ref) = refs
    h = h_ref[...]
    agg = p_ref[0] + p_ref[1]
    out = agg + (1.0 + eps_ref[0, 0]) * h
    hid = jnp.maximum(
        jnp.dot(out, w1_ref[...], preferred_element_type=jnp.float32)
        + b1_ref[...], 0.0)
    out2 = jnp.dot(hid, w2_ref[...],
                   preferred_element_type=jnp.float32) + b2_ref[...]
    mu = jnp.mean(out2, axis=-1, keepdims=True)
    var = jnp.mean((out2 - mu) ** 2, axis=-1, keepdims=True)
    out2 = (out2 - mu) * lax.rsqrt(var + 1e-5)
    if residual:
        out2 = out2 + h
    hn = jnp.where(out2 >= 0.0, out2, 0.1 * out2)
    hn_ref[...] = hn
    if emit_r:
        et = et_ref[...]
        for t in range(NUM_ET):
            r_ref[:, t, :] = jnp.maximum(hn + et[t:t + 1, :], 0.0)


def _layer(p, h, w1, b1, w2, b2, eps, et_next, din, residual):
    emit_r = et_next is not None
    grid = (N // BR,)
    full = lambda shp: pl.BlockSpec(shp, lambda i: (0,) * len(shp))
    in_specs = [
        pl.BlockSpec((2, BR, din), lambda i: (0, i, 0)),
        pl.BlockSpec((BR, din), lambda i: (i, 0)),
        full((din, HID)), full((1, HID)), full((HID, HID)), full((1, HID)),
        full((1, 1)),
    ]
    out_specs = [pl.BlockSpec((BR, HID), lambda i: (i, 0))]
    out_shape = [jax.ShapeDtypeStruct((N, HID), jnp.float32)]
    args = [p, h, w1, b1, w2, b2, eps]
    if emit_r:
        in_specs.append(full((NUM_ET, HID)))
        out_specs.append(pl.BlockSpec((BR, NUM_ET, HID), lambda i: (i, 0, 0)))
        out_shape.append(
            jax.ShapeDtypeStruct((N, NUM_ET, HID), jnp.float32))
        args.append(et_next)
    res = pl.pallas_call(
        functools.partial(_layer_body, din, residual, emit_r),
        grid=grid,
        in_specs=in_specs,
        out_specs=out_specs,
        out_shape=out_shape,
    )(*args)
    return res if emit_r else (res[0], None)


def _final_body(h3_ref, b8_ref, x45_ref, tp_ref, w1a_ref, w1b_ref, w1c_ref,
                b1_ref, w2_ref, b2_ref, out_ref):
    brow = b8_ref[0:1, :]
    iota = lax.broadcasted_iota(jnp.int32, (B, N), 0)
    oh = (iota == brow).astype(jnp.float32)
    g = jnp.dot(oh, h3_ref[...], preferred_element_type=jnp.float32)
    sums = jnp.dot(oh, x45_ref[...], preferred_element_type=jnp.float32)
    counts = jnp.sum(oh, axis=1, keepdims=True)
    denom = jnp.maximum(counts, 1.0)
    gs = jnp.concatenate(
        [counts, sums[:, 1:2] / denom, sums[:, 0:1] / denom,
         jnp.zeros((B, 5), jnp.float32)], axis=1)
    text = ((tp_ref[0] + tp_ref[1]) * (1.0 / L))[:, :TEXT]
    hid = (jnp.dot(g, w1a_ref[...], preferred_element_type=jnp.float32)
           + jnp.dot(gs, w1b_ref[...], preferred_element_type=jnp.float32)
           + jnp.dot(text, w1c_ref[...], preferred_element_type=jnp.float32)
           + b1_ref[...])
    hid = jnp.where(hid >= 0.0, hid, 0.1 * hid)
    out_ref[...] = jnp.dot(
        hid, w2_ref[...], preferred_element_type=jnp.float32) + b2_ref[...]


def _final(h3, b8, x45, tpart, w1a, w1b8, w1c, b1, w2, b2):
    return pl.pallas_call(
        _final_body,
        out_shape=jax.ShapeDtypeStruct((B, OUT), jnp.float32),
    )(h3, b8, x45, tpart, w1a, w1b8, w1c, b1, w2, b2)


NSUB = 16
K_CH = EPT // C_E
ROWS_PT = NROW // NSUB


def _edge_pass(r_tab, ridx3, dst3, zeros_nd, din):
    rf = r_tab.reshape(N * NUM_ET, din)
    mesh = plsc.VectorSubcoreMesh(core_axis_name="c", subcore_axis_name="s")

    @functools.partial(
        pl.kernel,
        out_type=jax.ShapeDtypeStruct((2, NROW, din), jnp.float32),
        mesh=mesh,
        scratch_types=[
            pltpu.VMEM((K_CH, C_E), jnp.int32),
            pltpu.VMEM((K_CH, C_E), jnp.int32),
            pltpu.VMEM((C_E, din), jnp.float32),
            pltpu.VMEM_SHARED((NROW, din), jnp.float32),
            pltpu.SemaphoreType.DMA,
        ],
    )
    def k(r_hbm, ridx_hbm, dst_hbm, z_hbm, out_hbm,
          ridx_v, dst_v, msg_v, acc, sem):
        c = lax.axis_index("c")
        s = lax.axis_index("s")
        wid = c * NSUB + s
        pltpu.sync_copy(z_hbm.at[pl.ds(s * ROWS_PT, ROWS_PT)],
                        acc.at[pl.ds(s * ROWS_PT, ROWS_PT)])
        pltpu.sync_copy(ridx_hbm.at[wid], ridx_v)
        pltpu.sync_copy(dst_hbm.at[wid], dst_v)
        plsc.subcore_barrier()

        def body(j, carry):
            pltpu.async_copy(r_hbm.at[ridx_v.at[j]], msg_v, sem).wait()
            pltpu.sync_copy(msg_v, acc.at[dst_v.at[j]], add=True)
            return carry

        lax.fori_loop(0, K_CH, body, 0)
        plsc.subcore_barrier()
        pltpu.sync_copy(acc.at[pl.ds(s * ROWS_PT, ROWS_PT)],
                        out_hbm.at[c].at[pl.ds(s * ROWS_PT, ROWS_PT)])

    return k(rf, ridx3, dst3, zeros_nd)


TOK_CH = (B * L) // (32 * 128)
BROWS_PT = 8


def _text_pass(token_embed, ids3, rep3, zeros_b):
    mesh = plsc.VectorSubcoreMesh(core_axis_name="c", subcore_axis_name="s")

    @functools.partial(
        pl.kernel,
        out_type=jax.ShapeDtypeStruct((2, B, HID), jnp.float32),
        mesh=mesh,
        scratch_types=[
            pltpu.VMEM((TOK_CH, 128), jnp.int32),
            pltpu.VMEM((TOK_CH, 128), jnp.int32),
            pltpu.VMEM((128, HID), jnp.float32),
            pltpu.VMEM_SHARED((B, HID), jnp.float32),
            pltpu.SemaphoreType.DMA,
        ],
    )
    def k(tok_hbm, ids_hbm, rep_hbm, z_hbm, out_hbm,
          ids_v, rep_v, msg_v, acc, sem):
        c = lax.axis_index("c")
        s = lax.axis_index("s")
        wid = c * NSUB + s

        @pl.when(s < B // BROWS_PT)
        def _():
            pltpu.sync_copy(z_hbm.at[pl.ds(s * BROWS_PT, BROWS_PT)],
                            acc.at[pl.ds(s * BROWS_PT, BROWS_PT)])

        pltpu.sync_copy(ids_hbm.at[wid], ids_v)
        pltpu.sync_copy(rep_hbm.at[wid], rep_v)
        plsc.subcore_barrier()
        for j in range(TOK_CH):
            pltpu.async_copy(tok_hbm.at[ids_v.at[j]], msg_v, sem).wait()
            pltpu.sync_copy(msg_v, acc.at[rep_v.at[j]], add=True)
        plsc.subcore_barrier()

        @pl.when(s < B // BROWS_PT)
        def _():
            pltpu.sync_copy(acc.at[pl.ds(s * BROWS_PT, BROWS_PT)],
                            out_hbm.at[c].at[pl.ds(s * BROWS_PT, BROWS_PT)])

    return k(token_embed, ids3, rep3, zeros_b)


def kernel(x, sql_mask, params, edge_index, edge_attr, batch, sql_ids):
    convs = params['convs']
    xp = jnp.pad(x, ((0, 0), (0, 3)))
    src = edge_index[0].astype(jnp.int32)
    dst = edge_index[1].astype(jnp.int32)
    ea = edge_attr.astype(jnp.int32)
    fill = (jnp.arange(EPAD - E, dtype=jnp.int32) * 97) % N
    src_pad = jnp.concatenate([src, fill])
    ea_pad = jnp.pad(ea, (0, EPAD - E))
    dst_pad = jnp.pad(dst, (0, EPAD - E), constant_values=N)

    wl0 = jnp.pad(convs[0]['Wl'], ((0, 0), (0, HID - 44)))
    bl0 = jnp.pad(convs[0]['bl'], (0, HID - 44)).reshape(1, HID)
    w1_0 = jnp.pad(convs[0]['W1'], ((0, HID - 44), (0, 0)))

    h0, r0, et0, et1, et2 = _prep(
        xp, params['op_embed'], params['edge_embed'],
        wl0, bl0,
        convs[1]['Wl'], convs[1]['bl'].reshape(1, HID),
        convs[2]['Wl'], convs[2]['bl'].reshape(1, HID))

    ridx3 = _ridx(src_pad, ea_pad).reshape(32, K_CH, C_E)
    dst3 = dst_pad.reshape(32, K_CH, C_E)

    ets = [et0, et1, et2]
    h = h0
    r_tab = r0
    dins = [HID, HID, HID]
    w1s = [w1_0, convs[1]['W1'], convs[2]['W1']]
    zeros_nd = jnp.zeros((NROW, HID), jnp.float32)
    for l in range(3):
        c = convs[l]
        p = _edge_pass(r_tab, ridx3, dst3, zeros_nd, dins[l])
        et_next = ets[l + 1] if l < 2 else None
        h, r_tab = _layer(
            p, h, w1s[l], c['b1'].reshape(1, HID),
            c['W2'], c['b2'].reshape(1, HID),
            c['eps'].reshape(1, 1), et_next, dins[l], residual=(l > 0))

    ids3 = sql_ids.astype(jnp.int32).reshape(32, TOK_CH, 128)
    rep3 = jnp.broadcast_to(
        jnp.arange(B, dtype=jnp.int32)[:, None], (B, L)).reshape(
            32, TOK_CH, 128)
    tok_pad = jnp.pad(params['token_embed'], ((0, 0), (0, HID - TEXT)))
    tpart = _text_pass(tok_pad, ids3, rep3,
                       jnp.zeros((B, HID), jnp.float32))

    b8 = jnp.broadcast_to(batch.astype(jnp.int32).reshape(1, N), (8, N))
    x45 = x[:, 4:6]
    mlp = params['mlp']
    w1a = mlp['W1'][:HID]
    w1b8 = jnp.pad(mlp['W1'][HID:HID + 3], ((0, 5), (0, 0)))
    w1c = mlp['W1'][HID + 3:]
    return _final(h, b8, x45, tpart, w1a, w1b8, w1c,
                  mlp['b1'].reshape(1, HID), mlp['W2'],
                  mlp['b2'].reshape(1, OUT))


# speedup vs baseline: 2.0837x; 1.0001x over previous
import functools

import jax
import jax.numpy as jnp
from jax import lax
from jax.experimental import pallas as pl
from jax.experimental.pallas import tpu as pltpu
from jax.experimental.pallas import tpu_sc as plsc

N = 10000
E = 320000
NUM_OP = 32
NUM_ET = 8
VOCAB = 100000
TEXT = 64
HID = 128
OUT = 512
B = 64
L = 128

BR = 1000
NROW = 10112
C_E = 128
EPT = 10112
EPAD = 32 * EPT


def _prep_body(xp_ref, ope_ref, ee_ref, wl0_ref, bl0_ref, wl1_ref, bl1_ref,
               wl2_ref, bl2_ref, h0_ref, r0_ref, et0_ref, et1_ref, et2_ref):
    op_ids = xp_ref[:, 0:1].astype(jnp.int32)
    iota = lax.broadcasted_iota(jnp.int32, (BR, NUM_OP), 1)
    oh = (iota == op_ids).astype(jnp.float32)
    emb = jnp.dot(oh, ope_ref[...], preferred_element_type=jnp.float32)
    h0 = jnp.concatenate(
        [emb, xp_ref[:, 1:13], jnp.zeros((BR, HID - 44), jnp.float32)], axis=1)
    h0_ref[...] = h0
    et0 = jnp.dot(ee_ref[...], wl0_ref[...],
                  preferred_element_type=jnp.float32) + bl0_ref[...]
    et1 = jnp.dot(ee_ref[...], wl1_ref[...],
                  preferred_element_type=jnp.float32) + bl1_ref[...]
    et2 = jnp.dot(ee_ref[...], wl2_ref[...],
                  preferred_element_type=jnp.float32) + bl2_ref[...]
    et0_ref[...] = et0
    et1_ref[...] = et1
    et2_ref[...] = et2
    for t in range(NUM_ET):
        r0_ref[:, t, :] = jnp.maximum(h0 + et0[t:t + 1, :], 0.0)


def _prep(xp, ope, ee, wl0, bl0, wl1, bl1, wl2, bl2):
    grid = (N // BR,)
    full = lambda shp: pl.BlockSpec(shp, lambda i: (0,) * len(shp))
    return pl.pallas_call(
        _prep_body,
        grid=grid,
        in_specs=[
            pl.BlockSpec((BR, 16), lambda i: (i, 0)),
            full((NUM_OP, 32)), full((NUM_ET, 16)),
            full((16, HID)), full((1, HID)),
            full((16, HID)), full((1, HID)),
            full((16, HID)), full((1, HID)),
        ],
        out_specs=[
            pl.BlockSpec((BR, HID), lambda i: (i, 0)),
            pl.BlockSpec((BR, NUM_ET, HID), lambda i: (i, 0, 0)),
            full((NUM_ET, HID)), full((NUM_ET, HID)), full((NUM_ET, HID)),
        ],
        out_shape=[
            jax.ShapeDtypeStruct((N, HID), jnp.float32),
            jax.ShapeDtypeStruct((N, NUM_ET, HID), jnp.float32),
            jax.ShapeDtypeStruct((NUM_ET, HID), jnp.float32),
            jax.ShapeDtypeStruct((NUM_ET, HID), jnp.float32),
            jax.ShapeDtypeStruct((NUM_ET, HID), jnp.float32),
        ],
    )(xp, ope, ee, wl0, bl0, wl1, bl1, wl2, bl2)


def _ridx_body(src_ref, ea_ref, out_ref):
    out_ref[...] = src_ref[...] * NUM_ET + ea_ref[...]


def _ridx(src_pad, ea_pad):
    s2 = src_pad.reshape(EPAD // 128, 128)
    e2 = ea_pad.reshape(EPAD // 128, 128)
    out = pl.pallas_call(
        _ridx_body,
        out_shape=jax.ShapeDtypeStruct((EPAD // 128, 128), jnp.int32),
    )(s2, e2)
    return out


def _layer_body(din, residual, emit_r, *refs):
    if emit_r:
        (p_ref, h_ref, w1_ref, b1_ref, w2_ref, b2_ref, eps_ref, et_ref,
         hn_ref, r_ref) = refs
    else:
        (p_ref, h_ref, w1_ref, b1_ref, w2_ref, b2_ref, eps_ref,
         hn_ref) = refs
    h = h_ref[...]
    agg = p_ref[0] + p_ref[1]
    out = agg + (1.0 + eps_ref[0, 0]) * h
    hid = jnp.maximum(
        jnp.dot(out, w1_ref[...], preferred_element_type=jnp.float32)
        + b1_ref[...], 0.0)
    out2 = jnp.dot(hid, w2_ref[...],
                   preferred_element_type=jnp.float32) + b2_ref[...]
    mu = jnp.mean(out2, axis=-1, keepdims=True)
    var = jnp.mean((out2 - mu) ** 2, axis=-1, keepdims=True)
    out2 = (out2 - mu) * lax.rsqrt(var + 1e-5)
    if residual:
        out2 = out2 + h
    hn = jnp.where(out2 >= 0.0, out2, 0.1 * out2)
    hn_ref[...] = hn
    if emit_r:
        et = et_ref[...]
        for t in range(NUM_ET):
            r_ref[:, t, :] = jnp.maximum(hn + et[t:t + 1, :], 0.0)


def _layer(p, h, w1, b1, w2, b2, eps, et_next, din, residual):
    emit_r = et_next is not None
    grid = (N // BR,)
    full = lambda shp: pl.BlockSpec(shp, lambda i: (0,) * len(shp))
    in_specs = [
        pl.BlockSpec((2, BR, din), lambda i: (0, i, 0)),
        pl.BlockSpec((BR, din), lambda i: (i, 0)),
        full((din, HID)), full((1, HID)), full((HID, HID)), full((1, HID)),
        full((1, 1)),
    ]
    out_specs = [pl.BlockSpec((BR, HID), lambda i: (i, 0))]
    out_shape = [jax.ShapeDtypeStruct((N, HID), jnp.float32)]
    args = [p, h, w1, b1, w2, b2, eps]
    if emit_r:
        in_specs.append(full((NUM_ET, HID)))
        out_specs.append(pl.BlockSpec((BR, NUM_ET, HID), lambda i: (i, 0, 0)))
        out_shape.append(
            jax.ShapeDtypeStruct((N, NUM_ET, HID), jnp.float32))
        args.append(et_next)
    res = pl.pallas_call(
        functools.partial(_layer_body, din, residual, emit_r),
        grid=grid,
        in_specs=in_specs,
        out_specs=out_specs,
        out_shape=out_shape,
    )(*args)
    return res if emit_r else (res[0], None)


def _final_body(h3_ref, b8_ref, x45_ref, tp_ref, w1a_ref, w1b_ref, w1c_ref,
                b1_ref, w2_ref, b2_ref, out_ref):
    brow = b8_ref[0:1, :]
    iota = lax.broadcasted_iota(jnp.int32, (B, N), 0)
    oh = (iota == brow).astype(jnp.float32)
    g = jnp.dot(oh, h3_ref[...], preferred_element_type=jnp.float32)
    sums = jnp.dot(oh, x45_ref[...], preferred_element_type=jnp.float32)
    counts = jnp.sum(oh, axis=1, keepdims=True)
    denom = jnp.maximum(counts, 1.0)
    gs = jnp.concatenate(
        [counts, sums[:, 1:2] / denom, sums[:, 0:1] / denom,
         jnp.zeros((B, 5), jnp.float32)], axis=1)
    text = ((tp_ref[0] + tp_ref[1]) * (1.0 / L))[:, :TEXT]
    hid = (jnp.dot(g, w1a_ref[...], preferred_element_type=jnp.float32)
           + jnp.dot(gs, w1b_ref[...], preferred_element_type=jnp.float32)
           + jnp.dot(text, w1c_ref[...], preferred_element_type=jnp.float32)
           + b1_ref[...])
    hid = jnp.where(hid >= 0.0, hid, 0.1 * hid)
    out_ref[...] = jnp.dot(
        hid, w2_ref[...], preferred_element_type=jnp.float32) + b2_ref[...]


def _final(h3, b8, x45, tpart, w1a, w1b8, w1c, b1, w2, b2):
    return pl.pallas_call(
        _final_body,
        out_shape=jax.ShapeDtypeStruct((B, OUT), jnp.float32),
    )(h3, b8, x45, tpart, w1a, w1b8, w1c, b1, w2, b2)


NSUB = 16
K_CH = EPT // C_E
ROWS_PT = NROW // NSUB


def _edge_pass(r_tab, ridx3, dst3, zeros_nd, din):
    rf = r_tab.reshape(N * NUM_ET, din)
    mesh = plsc.VectorSubcoreMesh(core_axis_name="c", subcore_axis_name="s")

    @functools.partial(
        pl.kernel,
        out_type=jax.ShapeDtypeStruct((2, NROW, din), jnp.float32),
        mesh=mesh,
        scratch_types=[
            pltpu.VMEM((K_CH, C_E), jnp.int32),
            pltpu.VMEM((K_CH, C_E), jnp.int32),
            pltpu.VMEM((C_E, din), jnp.float32),
            pltpu.VMEM_SHARED((NROW, din), jnp.float32),
            pltpu.SemaphoreType.DMA,
        ],
    )
    def k(r_hbm, ridx_hbm, dst_hbm, z_hbm, out_hbm,
          ridx_v, dst_v, msg_v, acc, sem):
        c = lax.axis_index("c")
        s = lax.axis_index("s")
        wid = c * NSUB + s
        pltpu.sync_copy(z_hbm.at[pl.ds(s * ROWS_PT, ROWS_PT)],
                        acc.at[pl.ds(s * ROWS_PT, ROWS_PT)])
        pltpu.sync_copy(ridx_hbm.at[wid], ridx_v)
        pltpu.sync_copy(dst_hbm.at[wid], dst_v)
        plsc.subcore_barrier()

        def body(j, carry):
            pltpu.async_copy(r_hbm.at[ridx_v.at[j]], msg_v, sem).wait()
            pltpu.sync_copy(msg_v, acc.at[dst_v.at[j]], add=True)
            return carry

        lax.fori_loop(0, K_CH, body, 0)
        plsc.subcore_barrier()
        pltpu.sync_copy(acc.at[pl.ds(s * ROWS_PT, ROWS_PT)],
                        out_hbm.at[c].at[pl.ds(s * ROWS_PT, ROWS_PT)])

    return k(rf, ridx3, dst3, zeros_nd)


TOK_CH = (B * L) // (32 * 128)
BROWS_PT = 8


def _text_pass(token_embed, ids3, rep3, zeros_b):
    mesh = plsc.VectorSubcoreMesh(core_axis_name="c", subcore_axis_name="s")

    @functools.partial(
        pl.kernel,
        out_type=jax.ShapeDtypeStruct((2, B, HID), jnp.float32),
        mesh=mesh,
        scratch_types=[
            pltpu.VMEM((TOK_CH, 128), jnp.int32),
            pltpu.VMEM((TOK_CH, 128), jnp.int32),
            pltpu.VMEM((128, HID), jnp.float32),
            pltpu.VMEM_SHARED((B, HID), jnp.float32),
            pltpu.SemaphoreType.DMA,
        ],
    )
    def k(tok_hbm, ids_hbm, rep_hbm, z_hbm, out_hbm,
          ids_v, rep_v, msg_v, acc, sem):
        c = lax.axis_index("c")
        s = lax.axis_index("s")
        wid = c * NSUB + s

        @pl.when(s < B // BROWS_PT)
        def _():
            pltpu.sync_copy(z_hbm.at[pl.ds(s * BROWS_PT, BROWS_PT)],
                            acc.at[pl.ds(s * BROWS_PT, BROWS_PT)])

        pltpu.sync_copy(ids_hbm.at[wid], ids_v)
        pltpu.sync_copy(rep_hbm.at[wid], rep_v)
        plsc.subcore_barrier()
        for j in range(TOK_CH):
            pltpu.async_copy(tok_hbm.at[ids_v.at[j]], msg_v, sem).wait()
            pltpu.sync_copy(msg_v, acc.at[rep_v.at[j]], add=True)
        plsc.subcore_barrier()

        @pl.when(s < B // BROWS_PT)
        def _():
            pltpu.sync_copy(acc.at[pl.ds(s * BROWS_PT, BROWS_PT)],
                            out_hbm.at[c].at[pl.ds(s * BROWS_PT, BROWS_PT)])

    return k(token_embed, ids3, rep3, zeros_b)


def kernel(x, sql_mask, params, edge_index, edge_attr, batch, sql_ids):
    convs = params['convs']
    xp = jnp.pad(x, ((0, 0), (0, 3)))
    src = edge_index[0].astype(jnp.int32)
    dst = edge_index[1].astype(jnp.int32)
    ea = edge_attr.astype(jnp.int32)
    fill = (jnp.arange(EPAD - E, dtype=jnp.int32) * 97) % N
    src_pad = jnp.concatenate([src, fill])
    ea_pad = jnp.pad(ea, (0, EPAD - E))
    fill_d = N + (jnp.arange(EPAD - E, dtype=jnp.int32) % (NROW - N))
    dst_pad = jnp.concatenate([dst, fill_d])

    wl0 = jnp.pad(convs[0]['Wl'], ((0, 0), (0, HID - 44)))
    bl0 = jnp.pad(convs[0]['bl'], (0, HID - 44)).reshape(1, HID)
    w1_0 = jnp.pad(convs[0]['W1'], ((0, HID - 44), (0, 0)))

    h0, r0, et0, et1, et2 = _prep(
        xp, params['op_embed'], params['edge_embed'],
        wl0, bl0,
        convs[1]['Wl'], convs[1]['bl'].reshape(1, HID),
        convs[2]['Wl'], convs[2]['bl'].reshape(1, HID))

    ridx3 = _ridx(src_pad, ea_pad).reshape(32, K_CH, C_E)
    dst3 = dst_pad.reshape(32, K_CH, C_E)

    ets = [et0, et1, et2]
    h = h0
    r_tab = r0
    dins = [HID, HID, HID]
    w1s = [w1_0, convs[1]['W1'], convs[2]['W1']]
    zeros_nd = jnp.zeros((NROW, HID), jnp.float32)
    for l in range(3):
        c = convs[l]
        p = _edge_pass(r_tab, ridx3, dst3, zeros_nd, dins[l])
        et_next = ets[l + 1] if l < 2 else None
        h, r_tab = _layer(
            p, h, w1s[l], c['b1'].reshape(1, HID),
            c['W2'], c['b2'].reshape(1, HID),
            c['eps'].reshape(1, 1), et_next, dins[l], residual=(l > 0))

    ids3 = sql_ids.astype(jnp.int32).reshape(32, TOK_CH, 128)
    rep3 = jnp.broadcast_to(
        jnp.arange(B, dtype=jnp.int32)[:, None], (B, L)).reshape(
            32, TOK_CH, 128)
    tok_pad = jnp.pad(params['token_embed'], ((0, 0), (0, HID - TEXT)))
    tpart = _text_pass(tok_pad, ids3, rep3,
                       jnp.zeros((B, HID), jnp.float32))

    b8 = jnp.broadcast_to(batch.astype(jnp.int32).reshape(1, N), (8, N))
    x45 = x[:, 4:6]
    mlp = params['mlp']
    w1a = mlp['W1'][:HID]
    w1b8 = jnp.pad(mlp['W1'][HID:HID + 3], ((0, 5), (0, 0)))
    w1c = mlp['W1'][HID + 3:]
    return _final(h, b8, x45, tpart, w1a, w1b8, w1c,
                  mlp['b1'].reshape(1, HID), mlp['W2'],
                  mlp['b2'].reshape(1, OUT))


# speedup vs baseline: 2.7563x; 1.3228x over previous
import functools

import jax
import jax.numpy as jnp
from jax import lax
from jax.experimental import pallas as pl
from jax.experimental.pallas import tpu as pltpu
from jax.experimental.pallas import tpu_sc as plsc

N = 10000
E = 320000
NUM_OP = 32
NUM_ET = 8
VOCAB = 100000
TEXT = 64
HID = 128
OUT = 512
B = 64
L = 128

BR = 1000
NROW = 10112
C_E = 128
G_E = 16
NGRP = 5
EPT = C_E * G_E * NGRP
EPAD = 32 * EPT


def _prep_body(xp_ref, ope_ref, ee_ref, wl0_ref, bl0_ref, wl1_ref, bl1_ref,
               wl2_ref, bl2_ref, h0_ref, r0_ref, et0_ref, et1_ref, et2_ref):
    op_ids = xp_ref[:, 0:1].astype(jnp.int32)
    iota = lax.broadcasted_iota(jnp.int32, (BR, NUM_OP), 1)
    oh = (iota == op_ids).astype(jnp.float32)
    emb = jnp.dot(oh, ope_ref[...], preferred_element_type=jnp.float32)
    h0 = jnp.concatenate(
        [emb, xp_ref[:, 1:13], jnp.zeros((BR, HID - 44), jnp.float32)], axis=1)
    h0_ref[...] = h0
    et0 = jnp.dot(ee_ref[...], wl0_ref[...],
                  preferred_element_type=jnp.float32) + bl0_ref[...]
    et1 = jnp.dot(ee_ref[...], wl1_ref[...],
                  preferred_element_type=jnp.float32) + bl1_ref[...]
    et2 = jnp.dot(ee_ref[...], wl2_ref[...],
                  preferred_element_type=jnp.float32) + bl2_ref[...]
    et0_ref[...] = et0
    et1_ref[...] = et1
    et2_ref[...] = et2
    for t in range(NUM_ET):
        r0_ref[:, t, :] = jnp.maximum(h0 + et0[t:t + 1, :], 0.0)


def _prep(xp, ope, ee, wl0, bl0, wl1, bl1, wl2, bl2):
    grid = (N // BR,)
    full = lambda shp: pl.BlockSpec(shp, lambda i: (0,) * len(shp))
    return pl.pallas_call(
        _prep_body,
        grid=grid,
        in_specs=[
            pl.BlockSpec((BR, 16), lambda i: (i, 0)),
            full((NUM_OP, 32)), full((NUM_ET, 16)),
            full((16, HID)), full((1, HID)),
            full((16, HID)), full((1, HID)),
            full((16, HID)), full((1, HID)),
        ],
        out_specs=[
            pl.BlockSpec((BR, HID), lambda i: (i, 0)),
            pl.BlockSpec((BR, NUM_ET, HID), lambda i: (i, 0, 0)),
            full((NUM_ET, HID)), full((NUM_ET, HID)), full((NUM_ET, HID)),
        ],
        out_shape=[
            jax.ShapeDtypeStruct((N, HID), jnp.float32),
            jax.ShapeDtypeStruct((N, NUM_ET, HID), jnp.float32),
            jax.ShapeDtypeStruct((NUM_ET, HID), jnp.float32),
            jax.ShapeDtypeStruct((NUM_ET, HID), jnp.float32),
            jax.ShapeDtypeStruct((NUM_ET, HID), jnp.float32),
        ],
    )(xp, ope, ee, wl0, bl0, wl1, bl1, wl2, bl2)


def _ridx_body(src_ref, ea_ref, out_ref):
    out_ref[...] = src_ref[...] * NUM_ET + ea_ref[...]


def _ridx(src_pad, ea_pad):
    s2 = src_pad.reshape(EPAD // 128, 128)
    e2 = ea_pad.reshape(EPAD // 128, 128)
    out = pl.pallas_call(
        _ridx_body,
        out_shape=jax.ShapeDtypeStruct((EPAD // 128, 128), jnp.int32),
    )(s2, e2)
    return out


def _layer_body(din, residual, emit_r, *refs):
    if emit_r:
        (p_ref, h_ref, w1_ref, b1_ref, w2_ref, b2_ref, eps_ref, et_ref,
         hn_ref, r_ref) = refs
    else:
        (p_ref, h_ref, w1_ref, b1_ref, w2_ref, b2_ref, eps_ref,
         hn_ref) = refs
    h = h_ref[...]
    agg = p_ref[0] + p_ref[1]
    out = agg + (1.0 + eps_ref[0, 0]) * h
    hid = jnp.maximum(
        jnp.dot(out, w1_ref[...], preferred_element_type=jnp.float32)
        + b1_ref[...], 0.0)
    out2 = jnp.dot(hid, w2_ref[...],
                   preferred_element_type=jnp.float32) + b2_ref[...]
    mu = jnp.mean(out2, axis=-1, keepdims=True)
    var = jnp.mean((out2 - mu) ** 2, axis=-1, keepdims=True)
    out2 = (out2 - mu) * lax.rsqrt(var + 1e-5)
    if residual:
        out2 = out2 + h
    hn = jnp.where(out2 >= 0.0, out2, 0.1 * out2)
    hn_ref[...] = hn
    if emit_r:
        et = et_ref[...]
        for t in range(NUM_ET):
            r_ref[:, t, :] = jnp.maximum(hn + et[t:t + 1, :], 0.0)


def _layer(p, h, w1, b1, w2, b2, eps, et_next, din, residual):
    emit_r = et_next is not None
    grid = (N // BR,)
    full = lambda shp: pl.BlockSpec(shp, lambda i: (0,) * len(shp))
    in_specs = [
        pl.BlockSpec((2, BR, din), lambda i: (0, i, 0)),
        pl.BlockSpec((BR, din), lambda i: (i, 0)),
        full((din, HID)), full((1, HID)), full((HID, HID)), full((1, HID)),
        full((1, 1)),
    ]
    out_specs = [pl.BlockSpec((BR, HID), lambda i: (i, 0))]
    out_shape = [jax.ShapeDtypeStruct((N, HID), jnp.float32)]
    args = [p, h, w1, b1, w2, b2, eps]
    if emit_r:
        in_specs.append(full((NUM_ET, HID)))
        out_specs.append(pl.BlockSpec((BR, NUM_ET, HID), lambda i: (i, 0, 0)))
        out_shape.append(
            jax.ShapeDtypeStruct((N, NUM_ET, HID), jnp.float32))
        args.append(et_next)
    res = pl.pallas_call(
        functools.partial(_layer_body, din, residual, emit_r),
        grid=grid,
        in_specs=in_specs,
        out_specs=out_specs,
        out_shape=out_shape,
    )(*args)
    return res if emit_r else (res[0], None)


def _final_body(h3_ref, b8_ref, x45_ref, tp_ref, w1a_ref, w1b_ref, w1c_ref,
                b1_ref, w2_ref, b2_ref, out_ref):
    brow = b8_ref[0:1, :]
    iota = lax.broadcasted_iota(jnp.int32, (B, N), 0)
    oh = (iota == brow).astype(jnp.float32)
    g = jnp.dot(oh, h3_ref[...], preferred_element_type=jnp.float32)
    sums = jnp.dot(oh, x45_ref[...], preferred_element_type=jnp.float32)
    counts = jnp.sum(oh, axis=1, keepdims=True)
    denom = jnp.maximum(counts, 1.0)
    gs = jnp.concatenate(
        [counts, sums[:, 1:2] / denom, sums[:, 0:1] / denom,
         jnp.zeros((B, 5), jnp.float32)], axis=1)
    text = ((tp_ref[0] + tp_ref[1]) * (1.0 / L))[:, :TEXT]
    hid = (jnp.dot(g, w1a_ref[...], preferred_element_type=jnp.float32)
           + jnp.dot(gs, w1b_ref[...], preferred_element_type=jnp.float32)
           + jnp.dot(text, w1c_ref[...], preferred_element_type=jnp.float32)
           + b1_ref[...])
    hid = jnp.where(hid >= 0.0, hid, 0.1 * hid)
    out_ref[...] = jnp.dot(
        hid, w2_ref[...], preferred_element_type=jnp.float32) + b2_ref[...]


def _final(h3, b8, x45, tpart, w1a, w1b8, w1c, b1, w2, b2):
    return pl.pallas_call(
        _final_body,
        out_shape=jax.ShapeDtypeStruct((B, OUT), jnp.float32),
    )(h3, b8, x45, tpart, w1a, w1b8, w1c, b1, w2, b2)


NSUB = 16
K_CH = EPT // C_E
ROWS_PT = NROW // NSUB


def _edge_pass(r_tab, ridx3, dst3, zeros_nd, din):
    rf = r_tab.reshape(N * NUM_ET, din)
    mesh = plsc.VectorSubcoreMesh(core_axis_name="c", subcore_axis_name="s")

    @functools.partial(
        pl.kernel,
        out_type=jax.ShapeDtypeStruct((2, NROW, din), jnp.float32),
        mesh=mesh,
        scratch_types=[
            pltpu.VMEM((2 * G_E, C_E), jnp.int32),
            pltpu.VMEM((2 * G_E, C_E), jnp.int32),
            pltpu.VMEM((C_E, din), jnp.float32),
            pltpu.VMEM((C_E, din), jnp.float32),
            pltpu.VMEM_SHARED((NROW, din), jnp.float32),
            pltpu.SemaphoreType.DMA,
            pltpu.SemaphoreType.DMA,
            pltpu.SemaphoreType.DMA,
            pltpu.SemaphoreType.DMA,
        ],
    )
    def k(r_hbm, ridx_hbm, dst_hbm, z_hbm, out_hbm,
          ridx_v, dst_v, msg0_v, msg1_v, acc, sem0, sem1, isem_r, isem_d):
        c = lax.axis_index("c")
        s = lax.axis_index("s")
        wid = c * NSUB + s
        pltpu.sync_copy(z_hbm.at[pl.ds(s * ROWS_PT, ROWS_PT)],
                        acc.at[pl.ds(s * ROWS_PT, ROWS_PT)])
        pltpu.sync_copy(ridx_hbm.at[wid].at[pl.ds(0, G_E)],
                        ridx_v.at[pl.ds(0, G_E)])
        pltpu.sync_copy(dst_hbm.at[wid].at[pl.ds(0, G_E)],
                        dst_v.at[pl.ds(0, G_E)])
        plsc.subcore_barrier()

        def group(g, carry):
            off = lax.rem(g, 2) * G_E
            nxt = lax.rem(g + 1, 2) * G_E

            @pl.when(g + 1 < NGRP)
            def _():
                pltpu.async_copy(
                    ridx_hbm.at[wid].at[pl.ds((g + 1) * G_E, G_E)],
                    ridx_v.at[pl.ds(nxt, G_E)], isem_r)
                pltpu.async_copy(
                    dst_hbm.at[wid].at[pl.ds((g + 1) * G_E, G_E)],
                    dst_v.at[pl.ds(nxt, G_E)], isem_d)

            pltpu.async_copy(r_hbm.at[ridx_v.at[off]], msg0_v, sem0)

            def body(t, carry2):
                j0 = off + 2 * t
                pltpu.async_copy(r_hbm.at[ridx_v.at[j0 + 1]], msg1_v, sem1)
                pltpu.make_async_copy(r_hbm.at[ridx_v.at[j0]], msg0_v,
                                      sem0).wait()
                pltpu.sync_copy(msg0_v, acc.at[dst_v.at[j0]], add=True)

                @pl.when(2 * t + 2 < G_E)
                def _():
                    pltpu.async_copy(r_hbm.at[ridx_v.at[j0 + 2]], msg0_v,
                                     sem0)

                pltpu.make_async_copy(r_hbm.at[ridx_v.at[j0 + 1]], msg1_v,
                                      sem1).wait()
                pltpu.sync_copy(msg1_v, acc.at[dst_v.at[j0 + 1]], add=True)
                return carry2

            lax.fori_loop(0, G_E // 2, body, 0)

            @pl.when(g + 1 < NGRP)
            def _():
                pltpu.make_async_copy(
                    ridx_hbm.at[wid].at[pl.ds(0, G_E)],
                    ridx_v.at[pl.ds(0, G_E)], isem_r).wait()
                pltpu.make_async_copy(
                    dst_hbm.at[wid].at[pl.ds(0, G_E)],
                    dst_v.at[pl.ds(0, G_E)], isem_d).wait()

            return carry

        lax.fori_loop(0, NGRP, group, 0)
        plsc.subcore_barrier()
        pltpu.sync_copy(acc.at[pl.ds(s * ROWS_PT, ROWS_PT)],
                        out_hbm.at[c].at[pl.ds(s * ROWS_PT, ROWS_PT)])

    return k(rf, ridx3, dst3, zeros_nd)


TOK_CH = (B * L) // (32 * 128)
BROWS_PT = 8


def _text_pass(token_embed, ids3, rep3, zeros_b):
    mesh = plsc.VectorSubcoreMesh(core_axis_name="c", subcore_axis_name="s")

    @functools.partial(
        pl.kernel,
        out_type=jax.ShapeDtypeStruct((2, B, HID), jnp.float32),
        mesh=mesh,
        scratch_types=[
            pltpu.VMEM((TOK_CH, 128), jnp.int32),
            pltpu.VMEM((TOK_CH, 128), jnp.int32),
            pltpu.VMEM((128, HID), jnp.float32),
            pltpu.VMEM_SHARED((B, HID), jnp.float32),
            pltpu.SemaphoreType.DMA,
        ],
    )
    def k(tok_hbm, ids_hbm, rep_hbm, z_hbm, out_hbm,
          ids_v, rep_v, msg_v, acc, sem):
        c = lax.axis_index("c")
        s = lax.axis_index("s")
        wid = c * NSUB + s

        @pl.when(s < B // BROWS_PT)
        def _():
            pltpu.sync_copy(z_hbm.at[pl.ds(s * BROWS_PT, BROWS_PT)],
                            acc.at[pl.ds(s * BROWS_PT, BROWS_PT)])

        pltpu.sync_copy(ids_hbm.at[wid], ids_v)
        pltpu.sync_copy(rep_hbm.at[wid], rep_v)
        plsc.subcore_barrier()
        for j in range(TOK_CH):
            pltpu.async_copy(tok_hbm.at[ids_v.at[j]], msg_v, sem).wait()
            pltpu.sync_copy(msg_v, acc.at[rep_v.at[j]], add=True)
        plsc.subcore_barrier()

        @pl.when(s < B // BROWS_PT)
        def _():
            pltpu.sync_copy(acc.at[pl.ds(s * BROWS_PT, BROWS_PT)],
                            out_hbm.at[c].at[pl.ds(s * BROWS_PT, BROWS_PT)])

    return k(token_embed, ids3, rep3, zeros_b)


def kernel(x, sql_mask, params, edge_index, edge_attr, batch, sql_ids):
    convs = params['convs']
    xp = jnp.pad(x, ((0, 0), (0, 3)))
    src = edge_index[0].astype(jnp.int32)
    dst = edge_index[1].astype(jnp.int32)
    ea = edge_attr.astype(jnp.int32)
    fill = (jnp.arange(EPAD - E, dtype=jnp.int32) * 97) % N
    src_pad = jnp.concatenate([src, fill])
    ea_pad = jnp.pad(ea, (0, EPAD - E))
    fill_d = N + (jnp.arange(EPAD - E, dtype=jnp.int32) % (NROW - N))
    dst_pad = jnp.concatenate([dst, fill_d])

    wl0 = jnp.pad(convs[0]['Wl'], ((0, 0), (0, HID - 44)))
    bl0 = jnp.pad(convs[0]['bl'], (0, HID - 44)).reshape(1, HID)
    w1_0 = jnp.pad(convs[0]['W1'], ((0, HID - 44), (0, 0)))

    h0, r0, et0, et1, et2 = _prep(
        xp, params['op_embed'], params['edge_embed'],
        wl0, bl0,
        convs[1]['Wl'], convs[1]['bl'].reshape(1, HID),
        convs[2]['Wl'], convs[2]['bl'].reshape(1, HID))

    ridx3 = _ridx(src_pad, ea_pad).reshape(32, K_CH, C_E)
    dst3 = dst_pad.reshape(32, K_CH, C_E)

    ets = [et0, et1, et2]
    h = h0
    r_tab = r0
    dins = [HID, HID, HID]
    w1s = [w1_0, convs[1]['W1'], convs[2]['W1']]
    zeros_nd = jnp.zeros((NROW, HID), jnp.float32)
    for l in range(3):
        c = convs[l]
        p = _edge_pass(r_tab, ridx3, dst3, zeros_nd, dins[l])
        et_next = ets[l + 1] if l < 2 else None
        h, r_tab = _layer(
            p, h, w1s[l], c['b1'].reshape(1, HID),
            c['W2'], c['b2'].reshape(1, HID),
            c['eps'].reshape(1, 1), et_next, dins[l], residual=(l > 0))

    ids3 = sql_ids.astype(jnp.int32).reshape(32, TOK_CH, 128)
    rep3 = jnp.broadcast_to(
        jnp.arange(B, dtype=jnp.int32)[:, None], (B, L)).reshape(
            32, TOK_CH, 128)
    tok_pad = jnp.pad(params['token_embed'], ((0, 0), (0, HID - TEXT)))
    tpart = _text_pass(tok_pad, ids3, rep3,
                       jnp.zeros((B, HID), jnp.float32))

    b8 = jnp.broadcast_to(batch.astype(jnp.int32).reshape(1, N), (8, N))
    x45 = x[:, 4:6]
    mlp = params['mlp']
    w1a = mlp['W1'][:HID]
    w1b8 = jnp.pad(mlp['W1'][HID:HID + 3], ((0, 5), (0, 0)))
    w1c = mlp['W1'][HID + 3:]
    return _final(h, b8, x45, tpart, w1a, w1b8, w1c,
                  mlp['b1'].reshape(1, HID), mlp['W2'],
                  mlp['b2'].reshape(1, OUT))
